# Initial kernel scaffold; baseline (speedup 1.0000x reference)
import jax
import jax.numpy as jnp
from jax import lax
from jax.experimental import pallas as pl
from jax.experimental.pallas import tpu as pltpu
from jax.experimental.pallas import tpu_sc as plsc

B, S, D_MODEL = 8, 2048, 2048
N_IMG, D_VIS = 576, 1024
OUT_LEN = S + N_IMG - 1
NTOK = B * S
NIMG_ROWS = B * N_IMG
NC, NS = 2, 16
NW = NC * NS
ROWS = B * OUT_LEN
ROWS_PAD = ROWS + NW
TXT_W = NTOK // NW
IMG_W = NIMG_ROWS // NW
CH = 16
TXT_CH = TXT_W // CH
IMG_CH = IMG_W // CH
W_PER_B = NW // B

_LANES = jnp.arange(16, dtype=jnp.int32)


def _mlp_body(x_ref, w1_ref, b1_ref, w2_ref, b2_ref, o_ref):
    h = jnp.dot(x_ref[...], w1_ref[...], preferred_element_type=jnp.float32)
    h = jax.nn.gelu(h + b1_ref[...])
    o_ref[...] = (
        jnp.dot(h, w2_ref[...], preferred_element_type=jnp.float32)
        + b2_ref[...]
    )


def _mlp(x, W1, b1, W2, b2):
    MB = 512
    return pl.pallas_call(
        _mlp_body,
        grid=(NIMG_ROWS // MB,),
        in_specs=[
            pl.BlockSpec((MB, D_VIS), lambda i: (i, 0)),
            pl.BlockSpec((D_VIS, D_MODEL), lambda i: (0, 0)),
            pl.BlockSpec((1, D_MODEL), lambda i: (0, 0)),
            pl.BlockSpec((D_MODEL, D_MODEL), lambda i: (0, 0)),
            pl.BlockSpec((1, D_MODEL), lambda i: (0, 0)),
        ],
        out_specs=pl.BlockSpec((MB, D_MODEL), lambda i: (i, 0)),
        out_shape=jax.ShapeDtypeStruct((NIMG_ROWS, D_MODEL), jnp.float32),
    )(x, W1, b1.reshape(1, D_MODEL), W2, b2.reshape(1, D_MODEL))


def _asm_body(ids_hbm, pos_hbm, img_hbm, tab_hbm, out_hbm,
              idsv, posv, dstv, dsti, rows, sem):
    c = lax.axis_index("c")
    s = lax.axis_index("s")
    wid = s * NC + c
    b = wid // W_PER_B
    q = wid - b * W_PER_B
    s0 = q * TXT_W
    row0 = b * OUT_LEN

    pltpu.sync_copy(ids_hbm.at[wid], idsv)
    pltpu.sync_copy(pos_hbm, posv)
    posb = plsc.load_gather(posv, [jnp.full((16,), b, jnp.int32)])
    padrow = jnp.full((16,), ROWS + wid, jnp.int32)

    def comp(g, carry):
        sv = s0 + g * CH + _LANES
        dst = jnp.where(
            sv < posb, row0 + sv,
            jnp.where(sv == posb, padrow, row0 + sv + (N_IMG - 1)))
        dstv[g, :] = dst
        return carry
    lax.fori_loop(0, TXT_CH, comp, 0)

    def txt(g, carry):
        pltpu.async_copy(tab_hbm.at[idsv.at[g]], rows, sem).wait()
        pltpu.async_copy(rows, out_hbm.at[dstv.at[g]], sem).wait()
        return carry
    lax.fori_loop(0, TXT_CH, txt, 0)

    def img(g, carry):
        off = q * IMG_W + g * CH
        dsti[0, :] = row0 + posb + off + _LANES
        pltpu.sync_copy(img_hbm.at[pl.ds(b * N_IMG + off, CH)], rows)
        pltpu.async_copy(rows, out_hbm.at[dsti.at[0]], sem).wait()
        return carry
    lax.fori_loop(0, IMG_CH, img, 0)


def _assemble(ids3, pos16, img_feats, table):
    mesh = plsc.VectorSubcoreMesh(
        core_axis_name="c", subcore_axis_name="s",
        num_cores=NC, num_subcores=NS)
    f = pl.kernel(
        _asm_body,
        out_type=jax.ShapeDtypeStruct((ROWS_PAD, D_MODEL), jnp.float32),
        mesh=mesh,
        scratch_types=[
            pltpu.VMEM((TXT_CH, CH), jnp.int32),
            pltpu.VMEM((16,), jnp.int32),
            pltpu.VMEM((TXT_CH, CH), jnp.int32),
            pltpu.VMEM((1, CH), jnp.int32),
            pltpu.VMEM((CH, D_MODEL), jnp.float32),
            pltpu.SemaphoreType.DMA,
        ],
    )
    return f(ids3, pos16, img_feats, table)


def kernel(input_ids, image_pos, images, embed_table, W1, b1, W2, b2):
    img_feats = _mlp(images.reshape(NIMG_ROWS, D_VIS), W1, b1, W2, b2)
    ids3 = input_ids.astype(jnp.int32).reshape(NW, TXT_CH, CH)
    pos16 = jnp.zeros((16,), jnp.int32).at[:B].set(image_pos.astype(jnp.int32))
    out_flat = _assemble(ids3, pos16, img_feats, embed_table)
    new_input_embeds = out_flat[:ROWS].reshape(B, OUT_LEN, D_MODEL)
    position_ids = jnp.broadcast_to(
        jnp.arange(OUT_LEN, dtype=jnp.int32), (B, OUT_LEN))
    attention_mask = jnp.ones((B, OUT_LEN), dtype=jnp.bool_)
    return new_input_embeds, position_ids, attention_mask


# speedup vs baseline: 3.7554x; 3.7554x over previous
import numpy as np

import jax
import jax.numpy as jnp
from jax import lax
from jax.experimental import pallas as pl
from jax.experimental.pallas import tpu as pltpu
from jax.experimental.pallas import tpu_sc as plsc

B, S, D_MODEL = 8, 2048, 2048
N_IMG, D_VIS = 576, 1024
OUT_LEN = S + N_IMG - 1
NTOK = B * S
NIMG_ROWS = B * N_IMG
NC, NS = 2, 16
NW = NC * NS
ROWS = B * OUT_LEN
ROWS_PAD = ROWS + NW
TXT_W = NTOK // NW
IMG_W = NIMG_ROWS // NW
CH = 16
TXT_CH = TXT_W // CH
IMG_CH = IMG_W // CH
W_PER_B = NW // B


def _mlp_body(x_ref, w1_ref, b1_ref, w2_ref, b2_ref, o_ref):
    h = jnp.dot(x_ref[...], w1_ref[...], preferred_element_type=jnp.float32)
    h = jax.nn.gelu(h + b1_ref[...])
    o_ref[...] = (
        jnp.dot(h, w2_ref[...], preferred_element_type=jnp.float32)
        + b2_ref[...]
    )


def _mlp(x, W1, b1, W2, b2):
    MB = 512
    return pl.pallas_call(
        _mlp_body,
        grid=(NIMG_ROWS // MB,),
        in_specs=[
            pl.BlockSpec((MB, D_VIS), lambda i: (i, 0)),
            pl.BlockSpec((D_VIS, D_MODEL), lambda i: (0, 0)),
            pl.BlockSpec((1, D_MODEL), lambda i: (0, 0)),
            pl.BlockSpec((D_MODEL, D_MODEL), lambda i: (0, 0)),
            pl.BlockSpec((1, D_MODEL), lambda i: (0, 0)),
        ],
        out_specs=pl.BlockSpec((MB, D_MODEL), lambda i: (i, 0)),
        out_shape=jax.ShapeDtypeStruct((NIMG_ROWS, D_MODEL), jnp.float32),
    )(x, W1, b1.reshape(1, D_MODEL), W2, b2.reshape(1, D_MODEL))


def _asm_body(ids_hbm, pos_hbm, img_hbm, tab_hbm, out_hbm,
              idsv, posv, dstv, dsti, rows, sem):
    c = lax.axis_index("c")
    s = lax.axis_index("s")
    wid = s * NC + c
    b = wid // W_PER_B
    q = wid - b * W_PER_B
    s0 = q * TXT_W
    row0 = b * OUT_LEN
    lanes = lax.iota(jnp.int32, 16)

    pltpu.sync_copy(ids_hbm.at[wid], idsv)
    pltpu.sync_copy(pos_hbm.at[wid], posv)
    posb = posv[...]
    padrow = jnp.full((16,), ROWS + wid, jnp.int32)

    def comp(g, carry):
        sv = s0 + g * CH + lanes
        dst = jnp.where(
            sv < posb, row0 + sv,
            jnp.where(sv == posb, padrow, row0 + sv + (N_IMG - 1)))
        dstv[g, :] = dst
        return carry
    lax.fori_loop(0, TXT_CH, comp, 0)

    def txt(g, carry):
        pltpu.async_copy(tab_hbm.at[idsv.at[g]], rows, sem).wait()
        pltpu.async_copy(rows, out_hbm.at[dstv.at[g]], sem).wait()
        return carry
    lax.fori_loop(0, TXT_CH, txt, 0)

    def img(g, carry):
        off = q * IMG_W + g * CH
        dsti[0, :] = row0 + posb + off + lanes
        pltpu.sync_copy(img_hbm.at[pl.ds(b * N_IMG + off, CH)], rows)
        pltpu.async_copy(rows, out_hbm.at[dsti.at[0]], sem).wait()
        return carry
    lax.fori_loop(0, IMG_CH, img, 0)


def _assemble(ids3, posw, img_feats, table):
    mesh = plsc.VectorSubcoreMesh(
        core_axis_name="c", subcore_axis_name="s",
        num_cores=NC, num_subcores=NS)
    f = pl.kernel(
        _asm_body,
        out_type=jax.ShapeDtypeStruct((ROWS_PAD, D_MODEL), jnp.float32),
        mesh=mesh,
        scratch_types=[
            pltpu.VMEM((TXT_CH, CH), jnp.int32),
            pltpu.VMEM((16,), jnp.int32),
            pltpu.VMEM((TXT_CH, CH), jnp.int32),
            pltpu.VMEM((1, CH), jnp.int32),
            pltpu.VMEM((CH, D_MODEL), jnp.float32),
            pltpu.SemaphoreType.DMA,
        ],
    )
    return f(ids3, posw, img_feats, table)


def kernel(input_ids, image_pos, images, embed_table, W1, b1, W2, b2):
    img_feats = _mlp(images.reshape(NIMG_ROWS, D_VIS), W1, b1, W2, b2)
    ids3 = input_ids.astype(jnp.int32).reshape(NW, TXT_CH, CH)
    posw = jnp.broadcast_to(
        jnp.repeat(image_pos.astype(jnp.int32), W_PER_B)[:, None], (NW, 16))
    out_flat = _assemble(ids3, posw, img_feats, embed_table)
    new_input_embeds = out_flat[:ROWS].reshape(B, OUT_LEN, D_MODEL)
    position_ids = jnp.broadcast_to(
        jnp.arange(OUT_LEN, dtype=jnp.int32), (B, OUT_LEN))
    attention_mask = jnp.ones((B, OUT_LEN), dtype=jnp.bool_)
    return new_input_embeds, position_ids, attention_mask


# speedup vs baseline: 6.3938x; 1.7026x over previous
import numpy as np

import jax
import jax.numpy as jnp
from jax import lax
from jax.experimental import pallas as pl
from jax.experimental.pallas import tpu as pltpu
from jax.experimental.pallas import tpu_sc as plsc

B, S, D_MODEL = 8, 2048, 2048
N_IMG, D_VIS = 576, 1024
OUT_LEN = S + N_IMG - 1
NTOK = B * S
NIMG_ROWS = B * N_IMG
NC, NS = 2, 16
NW = NC * NS
ROWS = B * OUT_LEN
ROWS_PAD = ROWS + NW
TXT_W = NTOK // NW
IMG_W = NIMG_ROWS // NW
CH = 16
TXT_CH = TXT_W // CH
IMG_CH = IMG_W // CH
W_PER_B = NW // B


def _mlp_body(x_ref, w1_ref, b1_ref, w2_ref, b2_ref, o_ref):
    h = jnp.dot(x_ref[...], w1_ref[...], preferred_element_type=jnp.float32)
    h = jax.nn.gelu(h + b1_ref[...])
    o_ref[...] = (
        jnp.dot(h, w2_ref[...], preferred_element_type=jnp.float32)
        + b2_ref[...]
    )


def _mlp(x, W1, b1, W2, b2):
    MB = 512
    return pl.pallas_call(
        _mlp_body,
        grid=(NIMG_ROWS // MB,),
        in_specs=[
            pl.BlockSpec((MB, D_VIS), lambda i: (i, 0)),
            pl.BlockSpec((D_VIS, D_MODEL), lambda i: (0, 0)),
            pl.BlockSpec((1, D_MODEL), lambda i: (0, 0)),
            pl.BlockSpec((D_MODEL, D_MODEL), lambda i: (0, 0)),
            pl.BlockSpec((1, D_MODEL), lambda i: (0, 0)),
        ],
        out_specs=pl.BlockSpec((MB, D_MODEL), lambda i: (i, 0)),
        out_shape=jax.ShapeDtypeStruct((NIMG_ROWS, D_MODEL), jnp.float32),
    )(x, W1, b1.reshape(1, D_MODEL), W2, b2.reshape(1, D_MODEL))


def _asm_body(ids_hbm, pos_hbm, img_hbm, tab_hbm, out_hbm,
              idsv, posv, dstv, dstiv, rows_a, rows_b, si_a, si_b, so_a, so_b):
    c = lax.axis_index("c")
    s = lax.axis_index("s")
    wid = s * NC + c
    b = wid // W_PER_B
    q = wid - b * W_PER_B
    s0 = q * TXT_W
    lanes = lax.iota(jnp.int32, 16)

    pltpu.sync_copy(ids_hbm.at[wid], idsv)
    pltpu.sync_copy(pos_hbm.at[wid], posv)
    posb = posv[...]

    padrow = posb + (q * IMG_W)
    def comp(g, carry):
        sv = s0 + g * CH + lanes
        dst = jnp.where(
            sv < posb, sv,
            jnp.where(sv == posb, padrow, sv + (N_IMG - 1)))
        dstv[g, :] = dst
        return carry
    lax.fori_loop(0, TXT_CH, comp, 0)

    bufs = (rows_a, rows_b)
    sins = (si_a, si_b)
    souts = (so_a, so_b)
    din = [None, None]
    dout = [None, None]
    din[0] = pltpu.async_copy(tab_hbm.at[idsv.at[0]], rows_a, si_a)
    for g in range(TXT_CH):
        p = g & 1
        if g >= 1:
            dout[1 - p].wait()
        if g + 1 < TXT_CH:
            din[1 - p] = pltpu.async_copy(
                tab_hbm.at[idsv.at[g + 1]], bufs[1 - p], sins[1 - p])
        din[p].wait()
        dout[p] = pltpu.async_copy(
            bufs[p], out_hbm.at[b].at[dstv.at[g]], souts[p])
    dout[(TXT_CH - 1) & 1].wait()

    def compi(g, carry):
        dstiv[g, :] = posb + q * IMG_W + g * CH + lanes
        return carry
    lax.fori_loop(0, IMG_CH, compi, 0)

    i0 = b * N_IMG + q * IMG_W
    din[0] = pltpu.async_copy(img_hbm.at[pl.ds(i0, CH)], rows_a, si_a)
    for g in range(IMG_CH):
        p = g & 1
        if g >= 1:
            dout[1 - p].wait()
        if g + 1 < IMG_CH:
            din[1 - p] = pltpu.async_copy(
                img_hbm.at[pl.ds(i0 + (g + 1) * CH, CH)],
                bufs[1 - p], sins[1 - p])
        din[p].wait()
        dout[p] = pltpu.async_copy(
            bufs[p], out_hbm.at[b].at[dstiv.at[g]], souts[p])
    dout[(IMG_CH - 1) & 1].wait()


def _assemble(ids3, posw, img_feats, table):
    mesh = plsc.VectorSubcoreMesh(
        core_axis_name="c", subcore_axis_name="s",
        num_cores=NC, num_subcores=NS)
    f = pl.kernel(
        _asm_body,
        out_type=jax.ShapeDtypeStruct((B, OUT_LEN, D_MODEL), jnp.float32),
        mesh=mesh,
        scratch_types=[
            pltpu.VMEM((TXT_CH, CH), jnp.int32),
            pltpu.VMEM((16,), jnp.int32),
            pltpu.VMEM((TXT_CH, CH), jnp.int32),
            pltpu.VMEM((IMG_CH, CH), jnp.int32),
            pltpu.VMEM((CH, D_MODEL), jnp.float32),
            pltpu.VMEM((CH, D_MODEL), jnp.float32),
            pltpu.SemaphoreType.DMA,
            pltpu.SemaphoreType.DMA,
            pltpu.SemaphoreType.DMA,
            pltpu.SemaphoreType.DMA,
        ],
    )
    return f(ids3, posw, img_feats, table)


def kernel(input_ids, image_pos, images, embed_table, W1, b1, W2, b2):
    img_feats = _mlp(images.reshape(NIMG_ROWS, D_VIS), W1, b1, W2, b2)
    ids3 = input_ids.astype(jnp.int32).reshape(NW, TXT_CH, CH)
    posw = jnp.broadcast_to(
        jnp.repeat(image_pos.astype(jnp.int32), W_PER_B)[:, None], (NW, 16))
    new_input_embeds = _assemble(ids3, posw, img_feats, embed_table)
    position_ids = jnp.broadcast_to(
        jnp.arange(OUT_LEN, dtype=jnp.int32), (B, OUT_LEN))
    attention_mask = jnp.ones((B, OUT_LEN), dtype=jnp.bool_)
    return new_input_embeds, position_ids, attention_mask


# speedup vs baseline: 10.3619x; 1.6206x over previous
import numpy as np

import jax
import jax.numpy as jnp
from jax import lax
from jax.experimental import pallas as pl
from jax.experimental.pallas import tpu as pltpu
from jax.experimental.pallas import tpu_sc as plsc

B, S, D_MODEL = 8, 2048, 2048
N_IMG, D_VIS = 576, 1024
OUT_LEN = S + N_IMG - 1
NTOK = B * S
NIMG_ROWS = B * N_IMG
NC, NS = 2, 16
NW = NC * NS
ROWS = B * OUT_LEN
ROWS_PAD = ROWS + NW
TXT_W = NTOK // NW
IMG_W = NIMG_ROWS // NW
CH = 16
TXT_CH = TXT_W // CH
IMG_CH = IMG_W // CH
W_PER_B = NW // B


def _mlp_body(x_ref, w1_ref, b1_ref, w2_ref, b2_ref, o_ref):
    h = jnp.dot(x_ref[...], w1_ref[...], preferred_element_type=jnp.float32)
    h = jax.nn.gelu(h + b1_ref[...])
    o_ref[...] = (
        jnp.dot(h, w2_ref[...], preferred_element_type=jnp.float32)
        + b2_ref[...]
    )


def _mlp(x, W1, b1, W2, b2):
    MB = 512
    return pl.pallas_call(
        _mlp_body,
        grid=(NIMG_ROWS // MB,),
        in_specs=[
            pl.BlockSpec((MB, D_VIS), lambda i: (i, 0)),
            pl.BlockSpec((D_VIS, D_MODEL), lambda i: (0, 0)),
            pl.BlockSpec((1, D_MODEL), lambda i: (0, 0)),
            pl.BlockSpec((D_MODEL, D_MODEL), lambda i: (0, 0)),
            pl.BlockSpec((1, D_MODEL), lambda i: (0, 0)),
        ],
        out_specs=pl.BlockSpec((MB, D_MODEL), lambda i: (i, 0)),
        out_shape=jax.ShapeDtypeStruct((NIMG_ROWS, D_MODEL), jnp.float32),
    )(x, W1, b1.reshape(1, D_MODEL), W2, b2.reshape(1, D_MODEL))


def _asm_body(ids_hbm, pos_hbm, img_hbm, tab_hbm, out_hbm,
              idsv, posv, dstv, dstiv, rows_a, rows_b, si_a, si_b, so_a, so_b):
    c = lax.axis_index("c")
    s = lax.axis_index("s")
    wid = s * NC + c
    b = wid // W_PER_B
    q = wid - b * W_PER_B
    s0 = q * TXT_W
    lanes = lax.iota(jnp.int32, 16)

    pltpu.sync_copy(ids_hbm.at[wid], idsv)
    pltpu.sync_copy(pos_hbm.at[wid], posv)
    posb = posv[...]

    padrow = posb + (q * IMG_W)
    def comp(g, carry):
        sv = s0 + g * CH + lanes
        j = jnp.where(
            sv < posb, sv,
            jnp.where(sv == posb, padrow, sv + (N_IMG - 1)))
        dstv[g, :] = j * B + b
        return carry
    lax.fori_loop(0, TXT_CH, comp, 0)

    bufs = (rows_a, rows_b)
    sins = (si_a, si_b)
    souts = (so_a, so_b)
    din = [None, None]
    dout = [None, None]
    din[0] = pltpu.async_copy(tab_hbm.at[idsv.at[0]], rows_a, si_a)
    for g in range(TXT_CH):
        p = g & 1
        if g >= 1:
            dout[1 - p].wait()
        if g + 1 < TXT_CH:
            din[1 - p] = pltpu.async_copy(
                tab_hbm.at[idsv.at[g + 1]], bufs[1 - p], sins[1 - p])
        din[p].wait()
        dout[p] = pltpu.async_copy(
            bufs[p], out_hbm.at[dstv.at[g]], souts[p])
    dout[(TXT_CH - 1) & 1].wait()

    def compi(g, carry):
        dstiv[g, :] = (posb + q * IMG_W + g * CH + lanes) * B + b
        return carry
    lax.fori_loop(0, IMG_CH, compi, 0)

    i0 = b * N_IMG + q * IMG_W
    din[0] = pltpu.async_copy(img_hbm.at[pl.ds(i0, CH)], rows_a, si_a)
    for g in range(IMG_CH):
        p = g & 1
        if g >= 1:
            dout[1 - p].wait()
        if g + 1 < IMG_CH:
            din[1 - p] = pltpu.async_copy(
                img_hbm.at[pl.ds(i0 + (g + 1) * CH, CH)],
                bufs[1 - p], sins[1 - p])
        din[p].wait()
        dout[p] = pltpu.async_copy(
            bufs[p], out_hbm.at[dstiv.at[g]], souts[p])
    dout[(IMG_CH - 1) & 1].wait()


def _assemble(ids3, posw, img_feats, table):
    mesh = plsc.VectorSubcoreMesh(
        core_axis_name="c", subcore_axis_name="s",
        num_cores=NC, num_subcores=NS)
    f = pl.kernel(
        _asm_body,
        out_type=jax.ShapeDtypeStruct((ROWS, D_MODEL), jnp.float32),
        mesh=mesh,
        scratch_types=[
            pltpu.VMEM((TXT_CH, CH), jnp.int32),
            pltpu.VMEM((16,), jnp.int32),
            pltpu.VMEM((TXT_CH, CH), jnp.int32),
            pltpu.VMEM((IMG_CH, CH), jnp.int32),
            pltpu.VMEM((CH, D_MODEL), jnp.float32),
            pltpu.VMEM((CH, D_MODEL), jnp.float32),
            pltpu.SemaphoreType.DMA,
            pltpu.SemaphoreType.DMA,
            pltpu.SemaphoreType.DMA,
            pltpu.SemaphoreType.DMA,
        ],
    )
    return f(ids3, posw, img_feats, table)


def kernel(input_ids, image_pos, images, embed_table, W1, b1, W2, b2):
    img_feats = _mlp(images.reshape(NIMG_ROWS, D_VIS), W1, b1, W2, b2)
    ids3 = input_ids.astype(jnp.int32).reshape(NW, TXT_CH, CH)
    posw = jnp.broadcast_to(
        jnp.repeat(image_pos.astype(jnp.int32), W_PER_B)[:, None], (NW, 16))
    out_jm = _assemble(ids3, posw, img_feats, embed_table)
    new_input_embeds = out_jm.reshape(OUT_LEN, B, D_MODEL).transpose(1, 0, 2)
    position_ids = jnp.broadcast_to(
        jnp.arange(OUT_LEN, dtype=jnp.int32), (B, OUT_LEN))
    attention_mask = jnp.ones((B, OUT_LEN), dtype=jnp.bool_)
    return new_input_embeds, position_ids, attention_mask


# speedup vs baseline: 13.1445x; 1.2685x over previous
import numpy as np

import jax
import jax.numpy as jnp
from jax import lax
from jax.experimental import pallas as pl
from jax.experimental.pallas import tpu as pltpu
from jax.experimental.pallas import tpu_sc as plsc

B, S, D_MODEL = 8, 2048, 2048
N_IMG, D_VIS = 576, 1024
OUT_LEN = S + N_IMG - 1
NTOK = B * S
NIMG_ROWS = B * N_IMG
NC, NS = 2, 16
NW = NC * NS
ROWS = B * OUT_LEN
ROWS_PAD = ROWS + NW
TXT_W = NTOK // NW
IMG_W = NIMG_ROWS // NW
CH = 16
TXT_CH = TXT_W // CH
IMG_CH = IMG_W // CH
W_PER_B = NW // B


def _mlp_body(x_ref, w1_ref, b1_ref, w2_ref, b2_ref, o_ref):
    h = jnp.dot(x_ref[...], w1_ref[...], preferred_element_type=jnp.float32)
    h = jax.nn.gelu(h + b1_ref[...])
    o_ref[...] = (
        jnp.dot(h, w2_ref[...], preferred_element_type=jnp.float32)
        + b2_ref[...]
    )


def _mlp(x, W1, b1, W2, b2):
    MB = 512
    return pl.pallas_call(
        _mlp_body,
        grid=(NIMG_ROWS // MB,),
        in_specs=[
            pl.BlockSpec((MB, D_VIS), lambda i: (i, 0)),
            pl.BlockSpec((D_VIS, D_MODEL), lambda i: (0, 0)),
            pl.BlockSpec((1, D_MODEL), lambda i: (0, 0)),
            pl.BlockSpec((D_MODEL, D_MODEL), lambda i: (0, 0)),
            pl.BlockSpec((1, D_MODEL), lambda i: (0, 0)),
        ],
        out_specs=pl.BlockSpec((MB, D_MODEL), lambda i: (i, 0)),
        out_shape=jax.ShapeDtypeStruct((NIMG_ROWS, D_MODEL), jnp.float32),
    )(x, W1, b1.reshape(1, D_MODEL), W2, b2.reshape(1, D_MODEL))


def _text_body(ids_hbm, pos_hbm, tab_hbm, out_hbm,
               idsv, posv, dstv, rows_a, rows_b, si_a, si_b, so_a, so_b):
    c = lax.axis_index("c")
    s = lax.axis_index("s")
    wid = s * NC + c
    b = wid // W_PER_B
    q = wid - b * W_PER_B
    s0 = q * TXT_W
    lanes = lax.iota(jnp.int32, 16)

    pltpu.sync_copy(ids_hbm.at[wid], idsv)
    pltpu.sync_copy(pos_hbm.at[wid], posv)
    posb = posv[...]

    padrow = posb + (q * IMG_W)
    def comp(g, carry):
        sv = s0 + g * CH + lanes
        j = jnp.where(
            sv < posb, sv,
            jnp.where(sv == posb, padrow, sv + (N_IMG - 1)))
        dstv[g, :] = j * B + b
        return carry
    lax.fori_loop(0, TXT_CH, comp, 0)

    bufs = (rows_a, rows_b)
    sins = (si_a, si_b)
    souts = (so_a, so_b)
    din = [None, None]
    dout = [None, None]
    din[0] = pltpu.async_copy(tab_hbm.at[idsv.at[0]], rows_a, si_a)
    for g in range(TXT_CH):
        p = g & 1
        if g >= 1:
            dout[1 - p].wait()
        if g + 1 < TXT_CH:
            din[1 - p] = pltpu.async_copy(
                tab_hbm.at[idsv.at[g + 1]], bufs[1 - p], sins[1 - p])
        din[p].wait()
        dout[p] = pltpu.async_copy(
            bufs[p], out_hbm.at[dstv.at[g]], souts[p])
    dout[(TXT_CH - 1) & 1].wait()


def _img_body(pos_hbm, img_hbm, out_hbm,
              posv, dstiv, rows_a, rows_b, si_a, si_b, so_a, so_b):
    c = lax.axis_index("c")
    s = lax.axis_index("s")
    wid = s * NC + c
    b = wid // W_PER_B
    q = wid - b * W_PER_B
    lanes = lax.iota(jnp.int32, 16)

    pltpu.sync_copy(pos_hbm.at[wid], posv)
    posb = posv[...]

    def compi(g, carry):
        dstiv[g, :] = (posb + q * IMG_W + g * CH + lanes) * B + b
        return carry
    lax.fori_loop(0, IMG_CH, compi, 0)

    bufs = (rows_a, rows_b)
    sins = (si_a, si_b)
    souts = (so_a, so_b)
    din = [None, None]
    dout = [None, None]
    i0 = b * N_IMG + q * IMG_W
    din[0] = pltpu.async_copy(img_hbm.at[pl.ds(i0, CH)], rows_a, si_a)
    for g in range(IMG_CH):
        p = g & 1
        if g >= 1:
            dout[1 - p].wait()
        if g + 1 < IMG_CH:
            din[1 - p] = pltpu.async_copy(
                img_hbm.at[pl.ds(i0 + (g + 1) * CH, CH)],
                bufs[1 - p], sins[1 - p])
        din[p].wait()
        dout[p] = pltpu.async_copy(
            bufs[p], out_hbm.at[dstiv.at[g]], souts[p])
    dout[(IMG_CH - 1) & 1].wait()


_MESH = plsc.VectorSubcoreMesh(
    core_axis_name="c", subcore_axis_name="s",
    num_cores=NC, num_subcores=NS)

_ROW_SCRATCH = [
    pltpu.VMEM((CH, D_MODEL), jnp.float32),
    pltpu.VMEM((CH, D_MODEL), jnp.float32),
    pltpu.SemaphoreType.DMA,
    pltpu.SemaphoreType.DMA,
    pltpu.SemaphoreType.DMA,
    pltpu.SemaphoreType.DMA,
]


def _assemble(ids3, posw, img_feats, table):
    text_f = pl.kernel(
        _text_body,
        out_type=(),
        mesh=_MESH,
        scratch_types=[
            pltpu.VMEM((TXT_CH, CH), jnp.int32),
            pltpu.VMEM((16,), jnp.int32),
            pltpu.VMEM((TXT_CH, CH), jnp.int32),
        ] + _ROW_SCRATCH,
    )
    img_f = pl.kernel(
        _img_body,
        out_type=(),
        mesh=_MESH,
        scratch_types=[
            pltpu.VMEM((16,), jnp.int32),
            pltpu.VMEM((IMG_CH, CH), jnp.int32),
        ] + _ROW_SCRATCH,
    )
    out_ref = jax.empty_ref(
        jax.ShapeDtypeStruct((ROWS, D_MODEL), jnp.float32))
    text_f(ids3, posw, table, out_ref)
    img_f(posw, img_feats, out_ref)
    return out_ref[...]


def kernel(input_ids, image_pos, images, embed_table, W1, b1, W2, b2):
    img_feats = _mlp(images.reshape(NIMG_ROWS, D_VIS), W1, b1, W2, b2)
    ids3 = input_ids.astype(jnp.int32).reshape(NW, TXT_CH, CH)
    posw = jnp.broadcast_to(
        jnp.repeat(image_pos.astype(jnp.int32), W_PER_B)[:, None], (NW, 16))
    out_jm = _assemble(ids3, posw, img_feats, embed_table)
    new_input_embeds = out_jm.reshape(OUT_LEN, B, D_MODEL).transpose(1, 0, 2)
    position_ids = jnp.broadcast_to(
        jnp.arange(OUT_LEN, dtype=jnp.int32), (B, OUT_LEN))
    attention_mask = jnp.ones((B, OUT_LEN), dtype=jnp.bool_)
    return new_input_embeds, position_ids, attention_mask


# speedup vs baseline: 13.3178x; 1.0132x over previous
import numpy as np

import jax
import jax.numpy as jnp
from jax import lax
from jax.experimental import pallas as pl
from jax.experimental.pallas import tpu as pltpu
from jax.experimental.pallas import tpu_sc as plsc

B, S, D_MODEL = 8, 2048, 2048
N_IMG, D_VIS = 576, 1024
OUT_LEN = S + N_IMG - 1
NTOK = B * S
NIMG_ROWS = B * N_IMG
NC, NS = 2, 16
NW = NC * NS
ROWS = B * OUT_LEN
ROWS_PAD = ROWS + NW
TXT_W = NTOK // NW
IMG_W = NIMG_ROWS // NW
CH = 16
TXT_CH = TXT_W // CH
IMG_CH = IMG_W // CH
W_PER_B = NW // B


def _mlp_body(x_ref, w1_ref, b1_ref, w2_ref, b2_ref, o_ref):
    h = jnp.dot(x_ref[...], w1_ref[...], preferred_element_type=jnp.float32)
    h = jax.nn.gelu(h + b1_ref[...])
    o_ref[...] = (
        jnp.dot(h, w2_ref[...], preferred_element_type=jnp.float32)
        + b2_ref[...]
    )


def _mlp(x, W1, b1, W2, b2):
    MB = 512
    return pl.pallas_call(
        _mlp_body,
        grid=(NIMG_ROWS // MB,),
        in_specs=[
            pl.BlockSpec((MB, D_VIS), lambda i: (i, 0)),
            pl.BlockSpec((D_VIS, D_MODEL), lambda i: (0, 0)),
            pl.BlockSpec((1, D_MODEL), lambda i: (0, 0)),
            pl.BlockSpec((D_MODEL, D_MODEL), lambda i: (0, 0)),
            pl.BlockSpec((1, D_MODEL), lambda i: (0, 0)),
        ],
        out_specs=pl.BlockSpec((MB, D_MODEL), lambda i: (i, 0)),
        out_shape=jax.ShapeDtypeStruct((NIMG_ROWS, D_MODEL), jnp.float32),
    )(x, W1, b1.reshape(1, D_MODEL), W2, b2.reshape(1, D_MODEL))


def _ring(n, gather_fn, scatter_fn, bufs, sins, souts):
    din = [None, None, None]
    dout = [None, None, None]
    din[0] = gather_fn(0, bufs[0], sins[0])
    din[1] = gather_fn(1, bufs[1], sins[1])
    for g in range(n):
        p = g % 3
        if g + 2 < n:
            if g >= 1:
                dout[(g + 2) % 3].wait()
            din[(g + 2) % 3] = gather_fn(
                g + 2, bufs[(g + 2) % 3], sins[(g + 2) % 3])
        din[p].wait()
        dout[p] = scatter_fn(g, bufs[p], souts[p])
    for k in (n - 3, n - 2, n - 1):
        dout[k % 3].wait()


def _text_body(ids_hbm, pos_hbm, tab_hbm, out_hbm,
               idsv, posv, dstv, rows_a, rows_b, rows_c,
               si_a, si_b, si_c, so_a, so_b, so_c):
    c = lax.axis_index("c")
    s = lax.axis_index("s")
    wid = s * NC + c
    b = wid // W_PER_B
    q = wid - b * W_PER_B
    s0 = q * TXT_W
    lanes = lax.iota(jnp.int32, 16)

    pltpu.sync_copy(ids_hbm.at[b, pl.ds(s0, TXT_W)], idsv)
    pltpu.sync_copy(pos_hbm.at[wid], posv)
    posb = posv[...]

    padrow = posb + (q * IMG_W)
    def comp(g, carry):
        sv = s0 + g * CH + lanes
        j = jnp.where(
            sv < posb, sv,
            jnp.where(sv == posb, padrow, sv + (N_IMG - 1)))
        dstv[g, :] = j * B + b
        return carry
    lax.fori_loop(0, TXT_CH, comp, 0)

    _ring(
        TXT_CH,
        lambda g, buf, sem: pltpu.async_copy(
            tab_hbm.at[idsv.at[pl.ds(g * CH, CH)]], buf, sem),
        lambda g, buf, sem: pltpu.async_copy(
            buf, out_hbm.at[dstv.at[g]], sem),
        (rows_a, rows_b, rows_c),
        (si_a, si_b, si_c),
        (so_a, so_b, so_c),
    )


def _img_body(pos_hbm, img_hbm, out_hbm,
              posv, dstiv, rows_a, rows_b, rows_c,
              si_a, si_b, si_c, so_a, so_b, so_c):
    c = lax.axis_index("c")
    s = lax.axis_index("s")
    wid = s * NC + c
    b = wid // W_PER_B
    q = wid - b * W_PER_B
    lanes = lax.iota(jnp.int32, 16)

    pltpu.sync_copy(pos_hbm.at[wid], posv)
    posb = posv[...]

    def compi(g, carry):
        dstiv[g, :] = (posb + q * IMG_W + g * CH + lanes) * B + b
        return carry
    lax.fori_loop(0, IMG_CH, compi, 0)

    i0 = b * N_IMG + q * IMG_W
    _ring(
        IMG_CH,
        lambda g, buf, sem: pltpu.async_copy(
            img_hbm.at[pl.ds(i0 + g * CH, CH)], buf, sem),
        lambda g, buf, sem: pltpu.async_copy(
            buf, out_hbm.at[dstiv.at[g]], sem),
        (rows_a, rows_b, rows_c),
        (si_a, si_b, si_c),
        (so_a, so_b, so_c),
    )


_MESH = plsc.VectorSubcoreMesh(
    core_axis_name="c", subcore_axis_name="s",
    num_cores=NC, num_subcores=NS)

_ROW_SCRATCH = [
    pltpu.VMEM((CH, D_MODEL), jnp.float32),
    pltpu.VMEM((CH, D_MODEL), jnp.float32),
    pltpu.VMEM((CH, D_MODEL), jnp.float32),
    pltpu.SemaphoreType.DMA,
    pltpu.SemaphoreType.DMA,
    pltpu.SemaphoreType.DMA,
    pltpu.SemaphoreType.DMA,
    pltpu.SemaphoreType.DMA,
    pltpu.SemaphoreType.DMA,
]


def _assemble(ids, posw, img_feats, table):
    text_f = pl.kernel(
        _text_body,
        out_type=(),
        mesh=_MESH,
        scratch_types=[
            pltpu.VMEM((TXT_W,), jnp.int32),
            pltpu.VMEM((16,), jnp.int32),
            pltpu.VMEM((TXT_CH, CH), jnp.int32),
        ] + _ROW_SCRATCH,
    )
    img_f = pl.kernel(
        _img_body,
        out_type=(),
        mesh=_MESH,
        scratch_types=[
            pltpu.VMEM((16,), jnp.int32),
            pltpu.VMEM((IMG_CH, CH), jnp.int32),
        ] + _ROW_SCRATCH,
    )
    out_ref = jax.empty_ref(
        jax.ShapeDtypeStruct((ROWS, D_MODEL), jnp.float32))
    text_f(ids, posw, table, out_ref)
    img_f(posw, img_feats, out_ref)
    return out_ref[...]


def kernel(input_ids, image_pos, images, embed_table, W1, b1, W2, b2):
    img_feats = _mlp(images.reshape(NIMG_ROWS, D_VIS), W1, b1, W2, b2)
    ids = input_ids.astype(jnp.int32)
    posw = jnp.broadcast_to(
        jnp.repeat(image_pos.astype(jnp.int32), W_PER_B)[:, None], (NW, 16))
    out_jm = _assemble(ids, posw, img_feats, embed_table)
    new_input_embeds = out_jm.reshape(OUT_LEN, B, D_MODEL).transpose(1, 0, 2)
    position_ids = jnp.broadcast_to(
        jnp.arange(OUT_LEN, dtype=jnp.int32), (B, OUT_LEN))
    attention_mask = jnp.ones((B, OUT_LEN), dtype=jnp.bool_)
    return new_input_embeds, position_ids, attention_mask


# speedup vs baseline: 13.3276x; 1.0007x over previous
import numpy as np

import jax
import jax.numpy as jnp
from jax import lax
from jax.experimental import pallas as pl
from jax.experimental.pallas import tpu as pltpu
from jax.experimental.pallas import tpu_sc as plsc

B, S, D_MODEL = 8, 2048, 2048
N_IMG, D_VIS = 576, 1024
OUT_LEN = S + N_IMG - 1
NTOK = B * S
NIMG_ROWS = B * N_IMG
NC, NS = 2, 16
NW = NC * NS
ROWS = B * OUT_LEN
ROWS_PAD = ROWS + NW
TXT_W = NTOK // NW
IMG_W = NIMG_ROWS // NW
CH = 16
TXT_CH = TXT_W // CH
IMG_CH = IMG_W // CH
W_PER_B = NW // B

_POS_IDS = np.broadcast_to(
    np.arange(OUT_LEN, dtype=np.int32), (B, OUT_LEN))
_ATTN_MASK = np.ones((B, OUT_LEN), dtype=np.bool_)


def _mlp_body(x_ref, w1_ref, b1_ref, w2_ref, b2_ref, o_ref):
    h = jnp.dot(x_ref[...], w1_ref[...], preferred_element_type=jnp.float32)
    h = jax.nn.gelu(h + b1_ref[...])
    o_ref[...] = (
        jnp.dot(h, w2_ref[...], preferred_element_type=jnp.float32)
        + b2_ref[...]
    )


def _mlp(x, W1, b1, W2, b2):
    MB = 512
    return pl.pallas_call(
        _mlp_body,
        grid=(NIMG_ROWS // MB,),
        in_specs=[
            pl.BlockSpec((MB, D_VIS), lambda i: (i, 0)),
            pl.BlockSpec((D_VIS, D_MODEL), lambda i: (0, 0)),
            pl.BlockSpec((1, D_MODEL), lambda i: (0, 0)),
            pl.BlockSpec((D_MODEL, D_MODEL), lambda i: (0, 0)),
            pl.BlockSpec((1, D_MODEL), lambda i: (0, 0)),
        ],
        out_specs=pl.BlockSpec((MB, D_MODEL), lambda i: (i, 0)),
        out_shape=jax.ShapeDtypeStruct((NIMG_ROWS, D_MODEL), jnp.float32),
    )(x, W1, b1.reshape(1, D_MODEL), W2, b2.reshape(1, D_MODEL))


def _ring(n, gather_fn, scatter_fn, bufs, sins, souts):
    din = [None, None, None]
    dout = [None, None, None]
    din[0] = gather_fn(0, bufs[0], sins[0])
    din[1] = gather_fn(1, bufs[1], sins[1])
    for g in range(n):
        p = g % 3
        if g + 2 < n:
            if g >= 1:
                dout[(g + 2) % 3].wait()
            din[(g + 2) % 3] = gather_fn(
                g + 2, bufs[(g + 2) % 3], sins[(g + 2) % 3])
        din[p].wait()
        dout[p] = scatter_fn(g, bufs[p], souts[p])
    for k in (n - 3, n - 2, n - 1):
        dout[k % 3].wait()


def _text_body(ids_hbm, pos_hbm, tab_hbm, out_hbm,
               idsv, posv, dstv, rows_a, rows_b, rows_c,
               si_a, si_b, si_c, so_a, so_b, so_c):
    c = lax.axis_index("c")
    s = lax.axis_index("s")
    wid = s * NC + c
    b = wid // W_PER_B
    q = wid - b * W_PER_B
    s0 = q * TXT_W
    lanes = lax.iota(jnp.int32, 16)

    pltpu.sync_copy(ids_hbm.at[b, pl.ds(s0, TXT_W)], idsv)
    pltpu.sync_copy(pos_hbm.at[wid], posv)
    posb = posv[...]

    padrow = posb + (q * IMG_W)
    def comp(g, carry):
        sv = s0 + g * CH + lanes
        j = jnp.where(
            sv < posb, sv,
            jnp.where(sv == posb, padrow, sv + (N_IMG - 1)))
        dstv[g, :] = j * B + b
        return carry
    lax.fori_loop(0, TXT_CH, comp, 0)

    _ring(
        TXT_CH,
        lambda g, buf, sem: pltpu.async_copy(
            tab_hbm.at[idsv.at[pl.ds(g * CH, CH)]], buf, sem),
        lambda g, buf, sem: pltpu.async_copy(
            buf, out_hbm.at[dstv.at[g]], sem),
        (rows_a, rows_b, rows_c),
        (si_a, si_b, si_c),
        (so_a, so_b, so_c),
    )


def _img_body(pos_hbm, img_hbm, out_hbm,
              posv, dstiv, rows_a, rows_b, rows_c,
              si_a, si_b, si_c, so_a, so_b, so_c):
    c = lax.axis_index("c")
    s = lax.axis_index("s")
    wid = s * NC + c
    b = wid // W_PER_B
    q = wid - b * W_PER_B
    lanes = lax.iota(jnp.int32, 16)

    pltpu.sync_copy(pos_hbm.at[wid], posv)
    posb = posv[...]

    def compi(g, carry):
        dstiv[g, :] = (posb + q * IMG_W + g * CH + lanes) * B + b
        return carry
    lax.fori_loop(0, IMG_CH, compi, 0)

    i0 = b * N_IMG + q * IMG_W
    _ring(
        IMG_CH,
        lambda g, buf, sem: pltpu.async_copy(
            img_hbm.at[pl.ds(i0 + g * CH, CH)], buf, sem),
        lambda g, buf, sem: pltpu.async_copy(
            buf, out_hbm.at[dstiv.at[g]], sem),
        (rows_a, rows_b, rows_c),
        (si_a, si_b, si_c),
        (so_a, so_b, so_c),
    )


_MESH = plsc.VectorSubcoreMesh(
    core_axis_name="c", subcore_axis_name="s",
    num_cores=NC, num_subcores=NS)

_ROW_SCRATCH = [
    pltpu.VMEM((CH, D_MODEL), jnp.float32),
    pltpu.VMEM((CH, D_MODEL), jnp.float32),
    pltpu.VMEM((CH, D_MODEL), jnp.float32),
    pltpu.SemaphoreType.DMA,
    pltpu.SemaphoreType.DMA,
    pltpu.SemaphoreType.DMA,
    pltpu.SemaphoreType.DMA,
    pltpu.SemaphoreType.DMA,
    pltpu.SemaphoreType.DMA,
]


def _assemble(ids, posw, img_feats, table):
    text_f = pl.kernel(
        _text_body,
        out_type=(),
        mesh=_MESH,
        scratch_types=[
            pltpu.VMEM((TXT_W,), jnp.int32),
            pltpu.VMEM((16,), jnp.int32),
            pltpu.VMEM((TXT_CH, CH), jnp.int32),
        ] + _ROW_SCRATCH,
    )
    img_f = pl.kernel(
        _img_body,
        out_type=(),
        mesh=_MESH,
        scratch_types=[
            pltpu.VMEM((16,), jnp.int32),
            pltpu.VMEM((IMG_CH, CH), jnp.int32),
        ] + _ROW_SCRATCH,
    )
    out_ref = jax.empty_ref(
        jax.ShapeDtypeStruct((ROWS, D_MODEL), jnp.float32))
    text_f(ids, posw, table, out_ref)
    img_f(posw, img_feats, out_ref)
    return out_ref[...]


def kernel(input_ids, image_pos, images, embed_table, W1, b1, W2, b2):
    img_feats = _mlp(images.reshape(NIMG_ROWS, D_VIS), W1, b1, W2, b2)
    ids = input_ids.astype(jnp.int32)
    posw = jnp.broadcast_to(
        jnp.repeat(image_pos.astype(jnp.int32), W_PER_B)[:, None], (NW, 16))
    out_jm = _assemble(ids, posw, img_feats, embed_table)
    new_input_embeds = out_jm.reshape(OUT_LEN, B, D_MODEL).transpose(1, 0, 2)
    position_ids = jnp.asarray(_POS_IDS)
    attention_mask = jnp.asarray(_ATTN_MASK)
    return new_input_embeds, position_ids, attention_mask


# speedup vs baseline: 13.3361x; 1.0006x over previous
import numpy as np

import jax
import jax.numpy as jnp
from jax import lax
from jax.experimental import pallas as pl
from jax.experimental.pallas import tpu as pltpu
from jax.experimental.pallas import tpu_sc as plsc

B, S, D_MODEL = 8, 2048, 2048
N_IMG, D_VIS = 576, 1024
OUT_LEN = S + N_IMG - 1
NTOK = B * S
NIMG_ROWS = B * N_IMG
NC, NS = 2, 16
NW = NC * NS
ROWS = B * OUT_LEN
ROWS_PAD = ROWS + NW
TXT_W = NTOK // NW
IMG_W = NIMG_ROWS // NW
CH = 16
TXT_CH = TXT_W // CH
IMG_CH = IMG_W // CH
W_PER_B = NW // B

_POS_IDS = np.broadcast_to(
    np.arange(OUT_LEN, dtype=np.int32), (B, OUT_LEN))
_ATTN_MASK = np.ones((B, OUT_LEN), dtype=np.bool_)


def _mlp_body(x_ref, w1_ref, b1_ref, w2_ref, b2_ref, o_ref):
    h = jnp.dot(x_ref[...], w1_ref[...], preferred_element_type=jnp.float32)
    h = jax.nn.gelu(h + b1_ref[...])
    o_ref[...] = (
        jnp.dot(h, w2_ref[...], preferred_element_type=jnp.float32)
        + b2_ref[...]
    )


def _mlp(x, W1, b1, W2, b2):
    MB = 512
    return pl.pallas_call(
        _mlp_body,
        grid=(NIMG_ROWS // MB,),
        in_specs=[
            pl.BlockSpec((MB, D_VIS), lambda i: (i, 0)),
            pl.BlockSpec((D_VIS, D_MODEL), lambda i: (0, 0)),
            pl.BlockSpec((1, D_MODEL), lambda i: (0, 0)),
            pl.BlockSpec((D_MODEL, D_MODEL), lambda i: (0, 0)),
            pl.BlockSpec((1, D_MODEL), lambda i: (0, 0)),
        ],
        out_specs=pl.BlockSpec((MB, D_MODEL), lambda i: (i, 0)),
        out_shape=jax.ShapeDtypeStruct((NIMG_ROWS, D_MODEL), jnp.float32),
    )(x, W1, b1.reshape(1, D_MODEL), W2, b2.reshape(1, D_MODEL))


def _ring(n, gather_fn, scatter_fn, bufs, sins, souts):
    din = [None, None, None]
    dout = [None, None, None]
    din[0] = gather_fn(0, bufs[0], sins[0])
    din[1] = gather_fn(1, bufs[1], sins[1])
    for g in range(n):
        p = g % 3
        if g + 2 < n:
            if g >= 1:
                dout[(g + 2) % 3].wait()
            din[(g + 2) % 3] = gather_fn(
                g + 2, bufs[(g + 2) % 3], sins[(g + 2) % 3])
        din[p].wait()
        dout[p] = scatter_fn(g, bufs[p], souts[p])
    for k in (n - 3, n - 2, n - 1):
        dout[k % 3].wait()


def _text_body(ids_hbm, pos_hbm, tab_hbm, out_hbm,
               idsv, posv, dstv, rows_a, rows_b, rows_c,
               si_a, si_b, si_c, so_a, so_b, so_c):
    c = lax.axis_index("c")
    s = lax.axis_index("s")
    wid = s * NC + c
    b = wid // W_PER_B
    q = wid - b * W_PER_B
    s0 = q * TXT_W
    lanes = lax.iota(jnp.int32, 16)

    pltpu.sync_copy(ids_hbm.at[b, pl.ds(s0, TXT_W)], idsv)
    pltpu.sync_copy(pos_hbm.at[wid], posv)
    posb = posv[...]

    padrow = posb + (q * IMG_W)
    def comp(g, carry):
        sv = s0 + g * CH + lanes
        j = jnp.where(
            sv < posb, sv,
            jnp.where(sv == posb, padrow, sv + (N_IMG - 1)))
        dstv[g, :] = j * B + b
        return carry
    lax.fori_loop(0, TXT_CH, comp, 0)

    _ring(
        TXT_CH,
        lambda g, buf, sem: pltpu.async_copy(
            tab_hbm.at[idsv.at[pl.ds(g * CH, CH)]], buf, sem),
        lambda g, buf, sem: pltpu.async_copy(
            buf, out_hbm.at[dstv.at[g]], sem),
        (rows_a, rows_b, rows_c),
        (si_a, si_b, si_c),
        (so_a, so_b, so_c),
    )


def _img_body(pos_hbm, img_hbm, out_hbm,
              posv, dstiv, rows_a, rows_b, rows_c,
              si_a, si_b, si_c, so_a, so_b, so_c):
    c = lax.axis_index("c")
    s = lax.axis_index("s")
    wid = s * NC + c
    b = wid // W_PER_B
    q = wid - b * W_PER_B
    lanes = lax.iota(jnp.int32, 16)

    pltpu.sync_copy(pos_hbm.at[wid], posv)
    posb = posv[...]

    def compi(g, carry):
        dstiv[g, :] = (posb + q * IMG_W + g * CH + lanes) * B + b
        return carry
    lax.fori_loop(0, IMG_CH, compi, 0)

    i0 = b * N_IMG + q * IMG_W
    _ring(
        IMG_CH,
        lambda g, buf, sem: pltpu.async_copy(
            img_hbm.at[pl.ds(i0 + g * CH, CH)], buf, sem),
        lambda g, buf, sem: pltpu.async_copy(
            buf, out_hbm.at[dstiv.at[g]], sem),
        (rows_a, rows_b, rows_c),
        (si_a, si_b, si_c),
        (so_a, so_b, so_c),
    )


def _mesh():
    return plsc.VectorSubcoreMesh(
        core_axis_name="c", subcore_axis_name="s",
        num_cores=NC, num_subcores=NS)


_ROW_SCRATCH = [
    pltpu.VMEM((CH, D_MODEL), jnp.float32),
    pltpu.VMEM((CH, D_MODEL), jnp.float32),
    pltpu.VMEM((CH, D_MODEL), jnp.float32),
    pltpu.SemaphoreType.DMA,
    pltpu.SemaphoreType.DMA,
    pltpu.SemaphoreType.DMA,
    pltpu.SemaphoreType.DMA,
    pltpu.SemaphoreType.DMA,
    pltpu.SemaphoreType.DMA,
]


def _assemble(ids, posw, img_feats, table):
    text_f = pl.kernel(
        _text_body,
        out_type=(),
        mesh=_mesh(),
        scratch_types=[
            pltpu.VMEM((TXT_W,), jnp.int32),
            pltpu.VMEM((16,), jnp.int32),
            pltpu.VMEM((TXT_CH, CH), jnp.int32),
        ] + _ROW_SCRATCH,
    )
    img_f = pl.kernel(
        _img_body,
        out_type=(),
        mesh=_mesh(),
        scratch_types=[
            pltpu.VMEM((16,), jnp.int32),
            pltpu.VMEM((IMG_CH, CH), jnp.int32),
        ] + _ROW_SCRATCH,
    )
    out_ref = jax.empty_ref(
        jax.ShapeDtypeStruct((ROWS, D_MODEL), jnp.float32))
    text_f(ids, posw, table, out_ref)
    img_f(posw, img_feats, out_ref)
    return out_ref[...]


def kernel(input_ids, image_pos, images, embed_table, W1, b1, W2, b2):
    img_feats = _mlp(images.reshape(NIMG_ROWS, D_VIS), W1, b1, W2, b2)
    ids = input_ids.astype(jnp.int32)
    posw = jnp.broadcast_to(
        jnp.repeat(image_pos.astype(jnp.int32), W_PER_B)[:, None], (NW, 16))
    out_jm = _assemble(ids, posw, img_feats, embed_table)
    new_input_embeds = out_jm.reshape(OUT_LEN, B, D_MODEL).transpose(1, 0, 2)
    position_ids = jnp.asarray(_POS_IDS)
    attention_mask = jnp.asarray(_ATTN_MASK)
    return new_input_embeds, position_ids, attention_mask
